# baseline (speedup 1.0000x reference)
import functools

import jax
import jax.numpy as jnp
from jax import lax
from jax.experimental import pallas as pl
from jax.experimental.pallas import tpu as pltpu
from jax.experimental.pallas import tpu_sc as plsc

_NB_OBJECTS = 8
_DIM_BODY = 10
_DIM_OBJECT = 15
_DIM_ACT = 4
_N_EDGES = 1024
_D_MP = 128
_BATCH = 256
_N_ISO = 2

_NW = 32
_B_PER_W = _BATCH // _NW
_ECH = 128
_NCH = _N_EDGES // _ECH
_LANES = 16
_DCH = _D_MP // _LANES
_NBUF = 2
_HCAP = _ECH // 2


def _seg_max_body(ef_hbm, ids_hbm, out_hbm, ids_v, xbuf, acc,
                  off_lo_s, off_hi_s, perm_s, sem0, sem1, semo):
    wid = lax.axis_index("s") * 2 + lax.axis_index("c")
    b0 = wid * _B_PER_W
    sems = (sem0, sem1)
    neg_inf = jnp.full((_LANES,), -jnp.inf, jnp.float32)

    pltpu.sync_copy(ids_hbm.at[pl.ds(b0, _B_PER_W)], ids_v)

    def batch_body(bi, _):
        b = b0 + bi
        ab = bi % 2
        @pl.when(bi >= 2)
        def _():
            pltpu.make_async_copy(acc.at[ab], out_hbm.at[b], semo).wait()
        for ci in range(min(_NBUF - 1, _NCH)):
            pltpu.async_copy(ef_hbm.at[b, pl.ds(ci * _ECH, _ECH)],
                             xbuf.at[ci % _NBUF], sems[ci % _NBUF])
        for ci in range(_NCH):
            if ci + _NBUF - 1 < _NCH:
                nc = ci + _NBUF - 1
                pltpu.async_copy(
                    ef_hbm.at[b, pl.ds(nc * _ECH, _ECH)],
                    xbuf.at[nc % _NBUF], sems[nc % _NBUF])
            xb = xbuf.at[ci % _NBUF]

            for j in range(_NB_OBJECTS):
                off_lo_s[j] = 0
                off_hi_s[j] = 0

            def scat_body(g, _):
                idv = ids_v[bi, pl.ds(ci * _ECH + g * _LANES, _LANES)]
                for k in range(_LANES):
                    idk = idv[k]
                    if k % 2 == 0:
                        p = off_lo_s[idk]
                        perm_s[idk * _ECH + p] = g * _LANES + k
                        off_lo_s[idk] = p + 1
                    else:
                        p = off_hi_s[idk]
                        perm_s[idk * _ECH + _HCAP + p] = g * _LANES + k
                        off_hi_s[idk] = p + 1
                return 0

            lax.fori_loop(0, _ECH // _LANES, scat_body, 0)

            pltpu.make_async_copy(
                ef_hbm.at[b, pl.ds(ci * _ECH, _ECH)], xb,
                sems[ci % _NBUF]).wait()

            def obj_body(j, _, ci=ci, xb=xb):
                if ci == 0:
                    accs = tuple(neg_inf for _ in range(_DCH))
                else:
                    accs = tuple(
                        acc[ab, j, pl.ds(c * _LANES, _LANES)]
                        for c in range(_DCH))

                def fold_body(pos, accs):
                    e = perm_s[pos]
                    return tuple(
                        jnp.maximum(accs[c], xb[e, pl.ds(c * _LANES, _LANES)])
                        for c in range(_DCH))

                accs = lax.fori_loop(
                    j * _ECH, j * _ECH + off_lo_s[j], fold_body, accs)
                accs = lax.fori_loop(
                    j * _ECH + _HCAP, j * _ECH + _HCAP + off_hi_s[j],
                    fold_body, accs)
                for c in range(_DCH):
                    acc[ab, j, pl.ds(c * _LANES, _LANES)] = accs[c]
                return 0

            lax.fori_loop(0, _NB_OBJECTS, obj_body, 0)

        pltpu.async_copy(acc.at[ab], out_hbm.at[b], semo)
        return 0

    lax.fori_loop(0, _B_PER_W, batch_body, 0)
    for t in range(2):
        bi = _B_PER_W - 2 + t
        pltpu.make_async_copy(
            acc.at[bi % 2], out_hbm.at[b0 + bi], semo).wait()


@functools.cache
def _get_seg_max():
    return functools.partial(
        pl.kernel,
        out_type=jax.ShapeDtypeStruct(
            (_BATCH, _NB_OBJECTS, _D_MP), jnp.float32),
        mesh=plsc.VectorSubcoreMesh(core_axis_name="c", subcore_axis_name="s"),
        scratch_types=[
            pltpu.VMEM((_B_PER_W, _N_EDGES), jnp.int32),
            pltpu.VMEM((_NBUF, _ECH, _D_MP), jnp.float32),
            pltpu.VMEM((2, _NB_OBJECTS, _D_MP), jnp.float32),
            pltpu.SMEM((_NB_OBJECTS,), jnp.int32),
            pltpu.SMEM((_NB_OBJECTS,), jnp.int32),
            pltpu.SMEM((_NB_OBJECTS * _ECH,), jnp.int32),
            pltpu.SemaphoreType.DMA,
            pltpu.SemaphoreType.DMA,
            pltpu.SemaphoreType.DMA,
        ],
    )(_seg_max_body)


def _mlp_body(obs_ref, act_ref, iso_ref, isof_ref, inc_ref,
              w1_ref, b1_ref, w2_ref, b2_ref,
              w3_ref, b3_ref, w4_ref, b4_ref,
              rho1_ref, rb1_ref, rho2_ref, rb2_ref,
              q1_ref, q2_ref):
    f32 = jnp.float32
    c0 = _DIM_BODY + _DIM_ACT
    c1 = c0 + _DIM_OBJECT

    def dot(a, b):
        return jnp.dot(a, b, preferred_element_type=f32)

    body_act = jnp.concatenate(
        [obs_ref[:, :_DIM_BODY], act_ref[...]], axis=1)
    base1 = dot(body_act, w1_ref[:c0]) + b1_ref[...][None, :]
    base2 = dot(body_act, w3_ref[:c0]) + b3_ref[...][None, :]
    o1 = jnp.zeros((_BATCH, 64), f32)
    o2 = jnp.zeros((_BATCH, 64), f32)
    for n in range(_NB_OBJECTS + _N_ISO):
        if n < _NB_OBJECTS:
            conn = obs_ref[:, _DIM_BODY + n * _DIM_OBJECT:
                           _DIM_BODY + (n + 1) * _DIM_OBJECT]
            feat = inc_ref[n]
        else:
            conn = iso_ref[:, n - _NB_OBJECTS, :]
            feat = isof_ref[:, n - _NB_OBJECTS, :]
        h1 = jnp.maximum(
            base1 + dot(conn, w1_ref[c0:c1]) + dot(feat, w1_ref[c1:]), 0.0)
        h2 = jnp.maximum(
            base2 + dot(conn, w3_ref[c0:c1]) + dot(feat, w3_ref[c1:]), 0.0)
        o1 = o1 + jnp.maximum(dot(h1, w2_ref[...]) + b2_ref[...][None, :],
                              0.0)
        o2 = o2 + jnp.maximum(dot(h2, w4_ref[...]) + b4_ref[...][None, :],
                              0.0)
    q1_ref[...] = dot(o1, rho1_ref[...]) + rb1_ref[...][None, :]
    q2_ref[...] = dot(o2, rho2_ref[...]) + rb2_ref[...][None, :]


def kernel(obs, act, edge_features, edges_to, isolated_nodes,
           isolated_nodes_features, phi_w1, phi_b1, phi_w2, phi_b2,
           phi_w3, phi_b3, phi_w4, phi_b4, rho_w1, rho_b1, rho_w2, rho_b2):
    inc = _get_seg_max()(edge_features, edges_to.astype(jnp.int32))

    inc_nodes = inc.reshape(_NB_OBJECTS, _BATCH, _D_MP)

    q1, q2 = pl.pallas_call(
        _mlp_body,
        out_shape=(
            jax.ShapeDtypeStruct((_BATCH, 1), jnp.float32),
            jax.ShapeDtypeStruct((_BATCH, 1), jnp.float32),
        ),
    )(obs, act, isolated_nodes, isolated_nodes_features, inc_nodes,
      phi_w1, phi_b1, phi_w2, phi_b2, phi_w3, phi_b3, phi_w4, phi_b4,
      rho_w1, rho_b1, rho_w2, rho_b2)
    return (q1, q2)


# speedup vs baseline: 1.0108x; 1.0108x over previous
import functools

import jax
import jax.numpy as jnp
from jax import lax
from jax.experimental import pallas as pl
from jax.experimental.pallas import tpu as pltpu
from jax.experimental.pallas import tpu_sc as plsc

_NB_OBJECTS = 8
_DIM_BODY = 10
_DIM_OBJECT = 15
_DIM_ACT = 4
_N_EDGES = 1024
_D_MP = 128
_BATCH = 256
_N_ISO = 2

_NW = 32
_B_PER_W = _BATCH // _NW
_ECH = 128
_NCH = _N_EDGES // _ECH
_LANES = 16
_DCH = _D_MP // _LANES
_NBUF = 2
_HCAP = _ECH // 2


def _seg_max_body(ef_hbm, ids_hbm, out_hbm, ids_v, xbuf, acc,
                  off_lo_s, off_hi_s, perm_s, sem0, sem1):
    wid = lax.axis_index("s") * 2 + lax.axis_index("c")
    b0 = wid * _B_PER_W
    sems = (sem0, sem1)
    neg_inf = jnp.full((_LANES,), -jnp.inf, jnp.float32)

    pltpu.sync_copy(ids_hbm.at[pl.ds(b0, _B_PER_W)], ids_v)

    def batch_body(bi, _):
        b = b0 + bi
        for ci in range(min(_NBUF - 1, _NCH)):
            pltpu.async_copy(ef_hbm.at[b, pl.ds(ci * _ECH, _ECH)],
                             xbuf.at[ci % _NBUF], sems[ci % _NBUF])
        for ci in range(_NCH):
            if ci + _NBUF - 1 < _NCH:
                nc = ci + _NBUF - 1
                pltpu.async_copy(
                    ef_hbm.at[b, pl.ds(nc * _ECH, _ECH)],
                    xbuf.at[nc % _NBUF], sems[nc % _NBUF])
            xb = xbuf.at[ci % _NBUF]

            for j in range(_NB_OBJECTS):
                off_lo_s[j] = 0
                off_hi_s[j] = 0

            def scat_body(g, _):
                idv = ids_v[bi, pl.ds(ci * _ECH + g * _LANES, _LANES)]
                for k in range(_LANES):
                    idk = idv[k]
                    if k % 2 == 0:
                        p = off_lo_s[idk]
                        perm_s[idk * _ECH + p] = g * _LANES + k
                        off_lo_s[idk] = p + 1
                    else:
                        p = off_hi_s[idk]
                        perm_s[idk * _ECH + _HCAP + p] = g * _LANES + k
                        off_hi_s[idk] = p + 1
                return 0

            lax.fori_loop(0, _ECH // _LANES, scat_body, 0)

            pltpu.make_async_copy(
                ef_hbm.at[b, pl.ds(ci * _ECH, _ECH)], xb,
                sems[ci % _NBUF]).wait()

            def obj_body(j, _, ci=ci, xb=xb):
                if ci == 0:
                    accs = tuple(neg_inf for _ in range(_DCH))
                else:
                    accs = tuple(
                        acc[j, pl.ds(c * _LANES, _LANES)]
                        for c in range(_DCH))

                def fold_body(pos, accs):
                    e = perm_s[pos]
                    return tuple(
                        jnp.maximum(accs[c], xb[e, pl.ds(c * _LANES, _LANES)])
                        for c in range(_DCH))

                accs = lax.fori_loop(
                    j * _ECH, j * _ECH + off_lo_s[j], fold_body, accs)
                accs = lax.fori_loop(
                    j * _ECH + _HCAP, j * _ECH + _HCAP + off_hi_s[j],
                    fold_body, accs)
                for c in range(_DCH):
                    acc[j, pl.ds(c * _LANES, _LANES)] = accs[c]
                return 0

            lax.fori_loop(0, _NB_OBJECTS, obj_body, 0)

        pltpu.sync_copy(acc, out_hbm.at[b])
        return 0

    lax.fori_loop(0, _B_PER_W, batch_body, 0)


@functools.cache
def _get_seg_max():
    return functools.partial(
        pl.kernel,
        out_type=jax.ShapeDtypeStruct(
            (_BATCH, _NB_OBJECTS, _D_MP), jnp.float32),
        mesh=plsc.VectorSubcoreMesh(core_axis_name="c", subcore_axis_name="s"),
        scratch_types=[
            pltpu.VMEM((_B_PER_W, _N_EDGES), jnp.int32),
            pltpu.VMEM((_NBUF, _ECH, _D_MP), jnp.float32),
            pltpu.VMEM((_NB_OBJECTS, _D_MP), jnp.float32),
            pltpu.SMEM((_NB_OBJECTS,), jnp.int32),
            pltpu.SMEM((_NB_OBJECTS,), jnp.int32),
            pltpu.SMEM((_NB_OBJECTS * _ECH,), jnp.int32),
            pltpu.SemaphoreType.DMA,
            pltpu.SemaphoreType.DMA,
        ],
    )(_seg_max_body)


def _mlp_body(obs_ref, act_ref, iso_ref, isof_ref, inc_ref,
              w1_ref, b1_ref, w2_ref, b2_ref,
              w3_ref, b3_ref, w4_ref, b4_ref,
              rho1_ref, rb1_ref, rho2_ref, rb2_ref,
              q1_ref, q2_ref):
    f32 = jnp.float32
    c0 = _DIM_BODY + _DIM_ACT
    c1 = c0 + _DIM_OBJECT

    def dot(a, b):
        return jnp.dot(a, b, preferred_element_type=f32)

    body_act = jnp.concatenate(
        [obs_ref[:, :_DIM_BODY], act_ref[...]], axis=1)
    base1 = dot(body_act, w1_ref[:c0]) + b1_ref[...][None, :]
    base2 = dot(body_act, w3_ref[:c0]) + b3_ref[...][None, :]
    o1 = jnp.zeros((_BATCH, 64), f32)
    o2 = jnp.zeros((_BATCH, 64), f32)
    for n in range(_NB_OBJECTS + _N_ISO):
        if n < _NB_OBJECTS:
            conn = obs_ref[:, _DIM_BODY + n * _DIM_OBJECT:
                           _DIM_BODY + (n + 1) * _DIM_OBJECT]
            feat = inc_ref[n]
        else:
            conn = iso_ref[:, n - _NB_OBJECTS, :]
            feat = isof_ref[:, n - _NB_OBJECTS, :]
        h1 = jnp.maximum(
            base1 + dot(conn, w1_ref[c0:c1]) + dot(feat, w1_ref[c1:]), 0.0)
        h2 = jnp.maximum(
            base2 + dot(conn, w3_ref[c0:c1]) + dot(feat, w3_ref[c1:]), 0.0)
        o1 = o1 + jnp.maximum(dot(h1, w2_ref[...]) + b2_ref[...][None, :],
                              0.0)
        o2 = o2 + jnp.maximum(dot(h2, w4_ref[...]) + b4_ref[...][None, :],
                              0.0)
    q1_ref[...] = dot(o1, rho1_ref[...]) + rb1_ref[...][None, :]
    q2_ref[...] = dot(o2, rho2_ref[...]) + rb2_ref[...][None, :]


def kernel(obs, act, edge_features, edges_to, isolated_nodes,
           isolated_nodes_features, phi_w1, phi_b1, phi_w2, phi_b2,
           phi_w3, phi_b3, phi_w4, phi_b4, rho_w1, rho_b1, rho_w2, rho_b2):
    inc = _get_seg_max()(edge_features, edges_to.astype(jnp.int32))

    inc_nodes = inc.reshape(_NB_OBJECTS, _BATCH, _D_MP)

    q1, q2 = pl.pallas_call(
        _mlp_body,
        out_shape=(
            jax.ShapeDtypeStruct((_BATCH, 1), jnp.float32),
            jax.ShapeDtypeStruct((_BATCH, 1), jnp.float32),
        ),
    )(obs, act, isolated_nodes, isolated_nodes_features, inc_nodes,
      phi_w1, phi_b1, phi_w2, phi_b2, phi_w3, phi_b3, phi_w4, phi_b4,
      rho_w1, rho_b1, rho_w2, rho_b2)
    return (q1, q2)
